# speedup vs baseline: 1.5685x; 1.5685x over previous
import functools

import jax
import jax.numpy as jnp
from jax import lax
from jax.experimental import pallas as pl
from jax.experimental.pallas import tpu as pltpu
from jax.experimental.pallas import tpu_sc as plsc

_NC = 2
_NS = 16
_NW = _NC * _NS


@functools.partial(jax.jit, static_argnums=(2, 3, 4))
def _sc_gather(table, idx, B, D, C):
    b_per_w = B // _NW
    n_chunks = b_per_w // C
    mesh = plsc.VectorSubcoreMesh(core_axis_name="c", subcore_axis_name="s")

    @functools.partial(
        pl.kernel,
        out_type=jax.ShapeDtypeStruct((B, D), jnp.float32),
        mesh=mesh,
        scratch_types=[
            pltpu.VMEM((C,), jnp.int32),
            pltpu.VMEM((C, D), jnp.float32),
            pltpu.SemaphoreType.DMA,
        ],
    )
    def k(table_hbm, idx_hbm, out_hbm, idx_v, rows_v, sem):
        wid = lax.axis_index("s") * _NC + lax.axis_index("c")
        base = wid * b_per_w

        def body(g, carry):
            off = base + g * C
            pltpu.sync_copy(idx_hbm.at[pl.ds(off, C)], idx_v)
            pltpu.async_copy(table_hbm.at[idx_v], rows_v, sem).wait()
            pltpu.sync_copy(rows_v, out_hbm.at[pl.ds(off, C)])
            return carry

        lax.fori_loop(0, n_chunks, body, 0)

    return k(table, idx)


def kernel(molatom_thing, real_atoms):
    m, a = molatom_thing.shape[0], molatom_thing.shape[1]
    rest = molatom_thing.shape[2:]
    d = rest[0]
    table = molatom_thing.reshape(m * a, d)
    idx = real_atoms.astype(jnp.int32)
    B = idx.shape[0]
    return _sc_gather(table, idx, B, d, 512)


# speedup vs baseline: 1.6795x; 1.0707x over previous
import functools

import jax
import jax.numpy as jnp
from jax import lax
from jax.experimental import pallas as pl
from jax.experimental.pallas import tpu as pltpu
from jax.experimental.pallas import tpu_sc as plsc

_NC = 2
_NS = 16
_NW = _NC * _NS


@functools.partial(jax.jit, static_argnums=(2, 3, 4, 5))
def _sc_gather(table, idx, B, D, C, NBUF):
    b_per_w = B // _NW
    n_chunks = b_per_w // C
    mesh = plsc.VectorSubcoreMesh(core_axis_name="c", subcore_axis_name="s")

    @functools.partial(
        pl.kernel,
        out_type=jax.ShapeDtypeStruct((B, D), jnp.float32),
        mesh=mesh,
        scratch_types=[
            pltpu.VMEM((b_per_w,), jnp.int32),
            pltpu.VMEM((NBUF, C, D), jnp.float32),
            pltpu.SemaphoreType.DMA((NBUF,)),
            pltpu.SemaphoreType.DMA((NBUF,)),
        ],
    )
    def k(table_hbm, idx_hbm, out_hbm, idx_v, rows_v, gsem, osem):
        wid = lax.axis_index("s") * _NC + lax.axis_index("c")
        base = wid * b_per_w
        pltpu.sync_copy(idx_hbm.at[pl.ds(base, b_per_w)], idx_v)

        def start_gather(g, b):
            return pltpu.async_copy(
                table_hbm.at[idx_v.at[pl.ds(g * C, C)]], rows_v.at[b],
                gsem.at[b])

        def start_out(g, b):
            return pltpu.async_copy(
                rows_v.at[b], out_hbm.at[pl.ds(base + g * C, C)], osem.at[b])

        hg = [start_gather(b, b) for b in range(min(NBUF, n_chunks))]
        ho = [None] * NBUF
        for g in range(n_chunks):
            b = g % NBUF
            hg[b].wait()
            ho[b] = start_out(g, b)
            if g + NBUF < n_chunks:
                ho[b].wait()
                hg[b] = start_gather(g + NBUF, b)
        for g in range(max(0, n_chunks - NBUF), n_chunks):
            ho[g % NBUF].wait()

    return k(table, idx)


def kernel(molatom_thing, real_atoms):
    m, a = molatom_thing.shape[0], molatom_thing.shape[1]
    rest = molatom_thing.shape[2:]
    d = rest[0]
    table = molatom_thing.reshape(m * a, d)
    idx = real_atoms.astype(jnp.int32)
    B = idx.shape[0]
    return _sc_gather(table, idx, B, d, 256, 2)


# speedup vs baseline: 1.6968x; 1.0103x over previous
import functools

import jax
import jax.numpy as jnp
from jax import lax
from jax.experimental import pallas as pl
from jax.experimental.pallas import tpu as pltpu
from jax.experimental.pallas import tpu_sc as plsc

_NC = 2
_NS = 16
_NW = _NC * _NS


@functools.partial(jax.jit, static_argnums=(2, 3, 4, 5))
def _sc_gather(table, idx, B, D, C, NBUF):
    b_per_w = B // _NW
    n_chunks = b_per_w // C
    mesh = plsc.VectorSubcoreMesh(core_axis_name="c", subcore_axis_name="s")

    @functools.partial(
        pl.kernel,
        out_type=jax.ShapeDtypeStruct((B, D), jnp.float32),
        mesh=mesh,
        scratch_types=[
            pltpu.VMEM((b_per_w,), jnp.int32),
            pltpu.VMEM((NBUF, C, D), jnp.float32),
            pltpu.SemaphoreType.DMA((NBUF,)),
            pltpu.SemaphoreType.DMA((NBUF,)),
        ],
    )
    def k(table_hbm, idx_hbm, out_hbm, idx_v, rows_v, gsem, osem):
        wid = lax.axis_index("s") * _NC + lax.axis_index("c")
        base = wid * b_per_w
        pltpu.sync_copy(idx_hbm.at[pl.ds(base, b_per_w)], idx_v)

        def start_gather(g, b):
            return pltpu.async_copy(
                table_hbm.at[idx_v.at[pl.ds(g * C, C)]], rows_v.at[b],
                gsem.at[b])

        def start_out(g, b):
            return pltpu.async_copy(
                rows_v.at[b], out_hbm.at[pl.ds(base + g * C, C)], osem.at[b])

        hg = [start_gather(b, b) for b in range(min(NBUF, n_chunks))]
        ho = [None] * NBUF
        for g in range(n_chunks):
            b = g % NBUF
            hg[b].wait()
            ho[b] = start_out(g, b)
            if g + NBUF < n_chunks:
                ho[b].wait()
                hg[b] = start_gather(g + NBUF, b)
        for g in range(max(0, n_chunks - NBUF), n_chunks):
            ho[g % NBUF].wait()

    return k(table, idx)


def kernel(molatom_thing, real_atoms):
    m, a = molatom_thing.shape[0], molatom_thing.shape[1]
    rest = molatom_thing.shape[2:]
    d = rest[0]
    table = molatom_thing.reshape(m * a, d)
    idx = real_atoms.astype(jnp.int32)
    B = idx.shape[0]
    return _sc_gather(table, idx, B, d, 256, 3)
